# baseline (speedup 1.0000x reference)
import functools

import jax
import jax.numpy as jnp
from jax import lax
from jax.experimental import pallas as pl


FREEZE_RATIO = 0.9
BLOCK_S = 512


def _f32_to_f16(x):
    u = lax.bitcast_convert_type(x, jnp.int32)
    sign = jnp.right_shift(u, 16) & 0x8000
    mag = u & 0x7FFFFFFF

    magr = mag + 0xFFF + (jnp.right_shift(mag, 13) & 1)
    h_norm = jnp.right_shift(magr, 13) - (112 << 10)
    h_norm = jnp.minimum(h_norm, 0x7C00)

    m = (mag & 0x7FFFFF) | 0x800000
    s = jnp.clip(126 - jnp.right_shift(mag, 23), 13, 30)
    half = jnp.left_shift(1, s - 1)
    h_den = jnp.right_shift(m + half - 1 + (jnp.right_shift(m, s) & 1), s)

    h = jnp.where(mag >= 0x38800000, h_norm, h_den)
    h = jnp.where(mag > 0x7F800000, 0x7E00, h)
    h = h | sign
    return h.astype(jnp.int16)


def _freeze_block_kernel(x_ref, frozen_ref, backup_ref, *, cutoff, n_backup_blocks):
    s = pl.program_id(1)
    row0 = s * BLOCK_S
    rows = jax.lax.broadcasted_iota(jnp.int32, (BLOCK_S, 1), 0) + row0
    keep = rows >= cutoff
    x = x_ref[...]
    frozen_ref[...] = jnp.where(keep[None], x, jnp.zeros((), dtype=x.dtype))

    @pl.when(s < n_backup_blocks)
    def _():
        backup_ref[...] = _f32_to_f16(x)


def kernel(tokens):
    batch, seq, d = tokens.shape
    cutoff = int(seq * FREEZE_RATIO)
    n_s_blocks = seq // BLOCK_S
    n_backup_blocks = pl.cdiv(cutoff, BLOCK_S)

    frozen, backup3 = pl.pallas_call(
        functools.partial(
            _freeze_block_kernel, cutoff=cutoff, n_backup_blocks=n_backup_blocks
        ),
        grid=(batch, n_s_blocks),
        in_specs=[
            pl.BlockSpec((1, BLOCK_S, d), lambda b, s: (b, s, 0)),
        ],
        out_specs=[
            pl.BlockSpec((1, BLOCK_S, d), lambda b, s: (b, s, 0)),
            pl.BlockSpec(
                (1, BLOCK_S, d),
                lambda b, s: (b, jnp.minimum(s, n_backup_blocks - 1), 0),
            ),
        ],
        out_shape=[
            jax.ShapeDtypeStruct((batch, seq, d), tokens.dtype),
            jax.ShapeDtypeStruct((batch, cutoff, d), jnp.int16),
        ],
    )(tokens)

    frozen_count = jnp.array(batch * cutoff, dtype=jnp.int32)
    backup = lax.bitcast_convert_type(backup3, jnp.float16).reshape(batch * cutoff, d)
    return frozen, frozen_count, backup


# speedup vs baseline: 1.1922x; 1.1922x over previous
import functools

import jax
import jax.numpy as jnp
from jax import lax
from jax.experimental import pallas as pl


FREEZE_RATIO = 0.9
BLOCK_S = 512

_SCALE_TO_INF = 2.0 ** 112
_SCALE_TO_ZERO = 2.0 ** -110


def _f32_to_f16(x):
    w = lax.bitcast_convert_type(x, jnp.int32)
    base = (jnp.abs(x) * _SCALE_TO_INF) * _SCALE_TO_ZERO
    shl1 = w + w
    e = jnp.maximum(lax.shift_right_logical(shl1, 24), 0x71)
    magic = lax.bitcast_convert_type(lax.shift_left(e, 23) + 0x07800000, jnp.float32)
    bits = lax.bitcast_convert_type(magic + base, jnp.int32)
    nonsign = (lax.shift_right_logical(bits, 13) & 0x7C00) + (bits & 0x0FFF)
    sign = lax.shift_right_logical(w, 16) & 0x8000
    return (nonsign | sign).astype(jnp.int16)


def _freeze_block_kernel(x_ref, frozen_ref, backup_ref, *, cutoff, boundary):
    s = pl.program_id(1)

    @pl.when(s < boundary)
    def _():
        frozen_ref[...] = jnp.zeros_like(frozen_ref)
        backup_ref[...] = _f32_to_f16(x_ref[...])

    @pl.when(s == boundary)
    def _():
        x = x_ref[...]
        rows = lax.broadcasted_iota(jnp.int32, (BLOCK_S, 1), 0) + boundary * BLOCK_S
        keep = rows >= cutoff
        frozen_ref[...] = jnp.where(keep[None], x, jnp.zeros((), dtype=x.dtype))
        backup_ref[...] = _f32_to_f16(x)

    @pl.when(s > boundary)
    def _():
        frozen_ref[...] = x_ref[...]


def kernel(tokens):
    batch, seq, d = tokens.shape
    cutoff = int(seq * FREEZE_RATIO)
    n_s_blocks = seq // BLOCK_S
    boundary = cutoff // BLOCK_S
    n_backup_blocks = pl.cdiv(cutoff, BLOCK_S)

    frozen, backup3 = pl.pallas_call(
        functools.partial(_freeze_block_kernel, cutoff=cutoff, boundary=boundary),
        grid=(batch, n_s_blocks),
        in_specs=[
            pl.BlockSpec((1, BLOCK_S, d), lambda b, s: (b, s, 0)),
        ],
        out_specs=[
            pl.BlockSpec((1, BLOCK_S, d), lambda b, s: (b, s, 0)),
            pl.BlockSpec(
                (1, BLOCK_S, d),
                lambda b, s: (b, jnp.minimum(s, n_backup_blocks - 1), 0),
            ),
        ],
        out_shape=[
            jax.ShapeDtypeStruct((batch, seq, d), tokens.dtype),
            jax.ShapeDtypeStruct((batch, cutoff, d), jnp.int16),
        ],
    )(tokens)

    frozen_count = jnp.array(batch * cutoff, dtype=jnp.int32)
    backup = lax.bitcast_convert_type(backup3, jnp.float16).reshape(batch * cutoff, d)
    return frozen, frozen_count, backup


# speedup vs baseline: 1.2191x; 1.0225x over previous
import functools

import jax
import jax.numpy as jnp
from jax import lax
from jax.experimental import pallas as pl


FREEZE_RATIO = 0.9
BLOCK_S = 1024

_SCALE_TO_INF = 2.0 ** 112
_SCALE_TO_ZERO = 2.0 ** -110


def _f32_to_f16(x):
    w = lax.bitcast_convert_type(x, jnp.int32)
    base = (jnp.abs(x) * _SCALE_TO_INF) * _SCALE_TO_ZERO
    shl1 = w + w
    e = jnp.maximum(lax.shift_right_logical(shl1, 24), 0x71)
    magic = lax.bitcast_convert_type(lax.shift_left(e, 23) + 0x07800000, jnp.float32)
    bits = lax.bitcast_convert_type(magic + base, jnp.int32)
    nonsign = (lax.shift_right_logical(bits, 13) & 0x7C00) + (bits & 0x0FFF)
    sign = lax.shift_right_logical(w, 16) & 0x8000
    return (nonsign | sign).astype(jnp.int16)


def _freeze_block_kernel(x_ref, frozen_ref, backup_ref, *, cutoff, boundary):
    s = pl.program_id(1)

    @pl.when(s < boundary)
    def _():
        frozen_ref[...] = jnp.zeros_like(frozen_ref)
        backup_ref[...] = _f32_to_f16(x_ref[...])

    @pl.when(s == boundary)
    def _():
        x = x_ref[...]
        rows = lax.broadcasted_iota(jnp.int32, (BLOCK_S, 1), 0) + boundary * BLOCK_S
        keep = rows >= cutoff
        frozen_ref[...] = jnp.where(keep[None], x, jnp.zeros((), dtype=x.dtype))
        backup_ref[...] = _f32_to_f16(x)

    @pl.when(s > boundary)
    def _():
        frozen_ref[...] = x_ref[...]


def kernel(tokens):
    batch, seq, d = tokens.shape
    cutoff = int(seq * FREEZE_RATIO)
    n_s_blocks = seq // BLOCK_S
    boundary = cutoff // BLOCK_S
    n_backup_blocks = pl.cdiv(cutoff, BLOCK_S)

    frozen, backup3 = pl.pallas_call(
        functools.partial(_freeze_block_kernel, cutoff=cutoff, boundary=boundary),
        grid=(batch, n_s_blocks),
        in_specs=[
            pl.BlockSpec((1, BLOCK_S, d), lambda b, s: (b, s, 0)),
        ],
        out_specs=[
            pl.BlockSpec((1, BLOCK_S, d), lambda b, s: (b, s, 0)),
            pl.BlockSpec(
                (1, BLOCK_S, d),
                lambda b, s: (b, jnp.minimum(s, n_backup_blocks - 1), 0),
            ),
        ],
        out_shape=[
            jax.ShapeDtypeStruct((batch, seq, d), tokens.dtype),
            jax.ShapeDtypeStruct((batch, cutoff, d), jnp.int16),
        ],
    )(tokens)

    frozen_count = jnp.array(batch * cutoff, dtype=jnp.int32)
    backup = lax.bitcast_convert_type(backup3, jnp.float16).reshape(batch * cutoff, d)
    return frozen, frozen_count, backup
